# Initial kernel scaffold; baseline (speedup 1.0000x reference)
import functools

import jax
import jax.numpy as jnp
from jax import lax
from jax.experimental import pallas as pl
from jax.experimental.pallas import tpu as pltpu
from jax.experimental.pallas import tpu_sc as plsc

_N = 10000
_E = 320000
_D = 128
_ED = 16
_H = 256
_NC = 2
_NS = 16
_CH = 80
_EPT = _E // _NS
_NCHUNK = _EPT // _CH
_RPT = _N // _NS
_WB = 125


def _tables_body(x_ref, w_ref, o_ref):
    o_ref[...] = jnp.dot(x_ref[...], w_ref[...], preferred_element_type=jnp.float32)


def _node_tables(nf, Wab):
    return pl.pallas_call(
        _tables_body,
        grid=(4,),
        in_specs=[
            pl.BlockSpec((_N, _D), lambda j: (0, 0)),
            pl.BlockSpec((_D, _D), lambda j: (0, j)),
        ],
        out_specs=pl.BlockSpec((_N, _D), lambda j: (j, 0)),
        out_shape=jax.ShapeDtypeStruct((4 * _N, _D), jnp.float32),
    )(nf, Wab)


_BE = 4000


def _q_body(ef_ref, w_ref, b_ref, o_ref):
    o_ref[0] = (
        jnp.dot(ef_ref[...], w_ref[0], preferred_element_type=jnp.float32)
        + b_ref[0]
    )


def _edge_q(ef, W1c3, b13):
    return pl.pallas_call(
        _q_body,
        grid=(2, _E // _BE),
        in_specs=[
            pl.BlockSpec((_BE, _ED), lambda j, i: (i, 0)),
            pl.BlockSpec((1, _ED, _D), lambda j, i: (j, 0, 0)),
            pl.BlockSpec((1, 1, _D), lambda j, i: (j, 0, 0)),
        ],
        out_specs=pl.BlockSpec((1, _BE, _D), lambda j, i: (j, i, 0)),
        out_shape=jax.ShapeDtypeStruct((2, _E, _D), jnp.float32),
    )(ef, W1c3, b13)


def _sc_body(pp4, q2, srcr, dstr, s_out, cnt_out,
             srcv, dstgv, dstv, bufA, bufB, bufQ, ones_v, wbuf, cbuf,
             s_sh, cnt_sh, semA, semB, semQ):
    cid = lax.axis_index("c")
    sid = lax.axis_index("s")
    rowbase = sid * _NCHUNK

    pltpu.sync_copy(srcr.at[pl.ds(rowbase, _NCHUNK)], srcv)
    pltpu.sync_copy(dstr.at[pl.ds(rowbase, _NCHUNK)], dstgv)
    pltpu.sync_copy(dstr.at[pl.ds(rowbase, _NCHUNK)], dstv)
    offA = cid * _N
    offB = 2 * _N + cid * _N

    def adj(i, carry):
        for c in range(_CH // 16):
            s = pl.ds(c * 16, 16)
            srcv[i, s] = srcv[i, s] + offA
            dstgv[i, s] = dstgv[i, s] + offB
        return carry

    lax.fori_loop(0, _NCHUNK, adj, 0)

    zero = jnp.zeros((16,), jnp.float32)
    one = jnp.ones((16,), jnp.float32)

    def zf(i, carry):
        for c in range(_D // 16):
            wbuf[i, pl.ds(c * 16, 16)] = zero
        return carry

    lax.fori_loop(0, _WB, zf, 0)

    def zc(i, carry):
        cbuf[i, pl.ds(0, 16)] = zero
        return carry

    lax.fori_loop(0, _RPT, zc, 0)

    def of(i, carry):
        ones_v[i, pl.ds(0, 16)] = one
        return carry

    lax.fori_loop(0, _CH, of, 0)

    for q in range(_RPT // _WB):
        pltpu.sync_copy(wbuf, s_sh.at[pl.ds(sid * _RPT + q * _WB, _WB)])
    pltpu.sync_copy(cbuf, cnt_sh.at[pl.ds(sid * _RPT, _RPT)])
    plsc.subcore_barrier()

    qrowbase = cid * _E + sid * _EPT

    def body(j, carry):
        ca = pltpu.async_copy(pp4.at[srcv.at[j]], bufA, semA)
        cb = pltpu.async_copy(pp4.at[dstgv.at[j]], bufB, semB)
        cq = pltpu.async_copy(q2.at[pl.ds(qrowbase + j * _CH, _CH)], bufQ, semQ)
        ca.wait()
        cb.wait()
        cq.wait()

        def comp(i, c2):
            for c in range(_D // 16):
                s = pl.ds(c * 16, 16)
                bufA[i, s] = jnp.maximum(bufA[i, s] + bufB[i, s] + bufQ[i, s], 0.0)
            return c2

        lax.fori_loop(0, _CH, comp, 0)
        pltpu.sync_copy(bufA, s_sh.at[dstv.at[j]], add=True)
        pltpu.sync_copy(ones_v, cnt_sh.at[dstv.at[j]], add=True)
        return carry

    lax.fori_loop(0, _NCHUNK, body, 0)
    plsc.subcore_barrier()

    for q in range(_RPT // _WB):
        pltpu.sync_copy(s_sh.at[pl.ds(sid * _RPT + q * _WB, _WB)], wbuf)
        pltpu.sync_copy(
            wbuf, s_out.at[pl.ds(cid * _N + sid * _RPT + q * _WB, _WB)])
    pltpu.sync_copy(cnt_sh.at[pl.ds(sid * _RPT, _RPT)], cbuf)
    pltpu.sync_copy(cbuf, cnt_out.at[pl.ds(cid * _N + sid * _RPT, _RPT)])


def _sc_aggregate(pp4, q2, src2, dst2):
    mesh = plsc.VectorSubcoreMesh(core_axis_name="c", subcore_axis_name="s")
    return pl.kernel(
        _sc_body,
        out_type=[
            jax.ShapeDtypeStruct((2 * _N, _D), jnp.float32),
            jax.ShapeDtypeStruct((2 * _N, 16), jnp.float32),
        ],
        mesh=mesh,
        scratch_types=[
            pltpu.VMEM((_NCHUNK, _CH), jnp.int32),
            pltpu.VMEM((_NCHUNK, _CH), jnp.int32),
            pltpu.VMEM((_NCHUNK, _CH), jnp.int32),
            pltpu.VMEM((_CH, _D), jnp.float32),
            pltpu.VMEM((_CH, _D), jnp.float32),
            pltpu.VMEM((_CH, _D), jnp.float32),
            pltpu.VMEM((_CH, 16), jnp.float32),
            pltpu.VMEM((_WB, _D), jnp.float32),
            pltpu.VMEM((_RPT, 16), jnp.float32),
            pltpu.VMEM_SHARED((_N, _D), jnp.float32),
            pltpu.VMEM_SHARED((_N, 16), jnp.float32),
            pltpu.SemaphoreType.DMA,
            pltpu.SemaphoreType.DMA,
            pltpu.SemaphoreType.DMA,
        ],
    )(pp4, q2, src2, dst2)


_RB = 2000


def _gru_body(s0_ref, s1_ref, cnt_ref, h_ref, w2_ref, wih_ref, whh_ref,
              b2_ref, bih_ref, bhh_ref, o_ref):
    cnt = cnt_ref[...][:, 0:1]
    inv = 1.0 / jnp.maximum(cnt, 1.0)
    pres = cnt * inv
    w2 = w2_ref[...]
    aggs = (
        jnp.dot(s0_ref[...], w2[:_D], preferred_element_type=jnp.float32)
        + jnp.dot(s1_ref[...], w2[_D:], preferred_element_type=jnp.float32)
    )
    agg = aggs * inv + b2_ref[...] * pres
    dn = (((1,), (1,)), ((), ()))
    gi = lax.dot_general(agg, wih_ref[...], dn,
                         preferred_element_type=jnp.float32) + bih_ref[...]
    h = h_ref[...]
    gh = lax.dot_general(h, whh_ref[...], dn,
                         preferred_element_type=jnp.float32) + bhh_ref[...]
    r = jax.nn.sigmoid(gi[:, :_D] + gh[:, :_D])
    z = jax.nn.sigmoid(gi[:, _D:2 * _D] + gh[:, _D:2 * _D])
    nn = jnp.tanh(gi[:, 2 * _D:] + r * gh[:, 2 * _D:])
    o_ref[...] = (1.0 - z) * nn + z * h


def _gru_update(s0, s1, cnt16, nf, W2, b2, Wih, Whh, bih, bhh):
    return pl.pallas_call(
        _gru_body,
        grid=(_N // _RB,),
        in_specs=[
            pl.BlockSpec((_RB, _D), lambda i: (i, 0)),
            pl.BlockSpec((_RB, _D), lambda i: (i, 0)),
            pl.BlockSpec((_RB, 16), lambda i: (i, 0)),
            pl.BlockSpec((_RB, _D), lambda i: (i, 0)),
            pl.BlockSpec((_H, _H), lambda i: (0, 0)),
            pl.BlockSpec((3 * _D, _H), lambda i: (0, 0)),
            pl.BlockSpec((3 * _D, _D), lambda i: (0, 0)),
            pl.BlockSpec((1, _H), lambda i: (0, 0)),
            pl.BlockSpec((1, 3 * _D), lambda i: (0, 0)),
            pl.BlockSpec((1, 3 * _D), lambda i: (0, 0)),
        ],
        out_specs=pl.BlockSpec((_RB, _D), lambda i: (i, 0)),
        out_shape=jax.ShapeDtypeStruct((_N, _D), jnp.float32),
    )(s0, s1, cnt16, nf, W2, Wih, Whh, b2.reshape(1, _H),
      bih.reshape(1, 3 * _D), bhh.reshape(1, 3 * _D))


def kernel(node_feats, edge_feats, W1, b1, W2, b2, Wih, Whh, bih, bhh,
           edge_index):
    nf = node_feats
    Wab = jnp.concatenate([W1[:_D], W1[_D:2 * _D]], axis=1)
    W1c3 = W1[2 * _D:].reshape(_ED, 2, _D).transpose(1, 0, 2)
    b13 = b1.reshape(2, 1, _D)
    src2 = edge_index[0].astype(jnp.int32).reshape(_E // _CH, _CH)
    dst2 = edge_index[1].astype(jnp.int32).reshape(_E // _CH, _CH)

    pp4 = _node_tables(nf, Wab)
    q2 = _edge_q(edge_feats, W1c3, b13).reshape(2 * _E, _D)
    s2, cnt16 = _sc_aggregate(pp4, q2, src2, dst2)
    return _gru_update(s2[:_N], s2[_N:], cnt16[:_N], nf,
                       W2, b2, Wih, Whh, bih, bhh)


# speedup vs baseline: 1.9392x; 1.9392x over previous
import functools

import jax
import jax.numpy as jnp
from jax import lax
from jax.experimental import pallas as pl
from jax.experimental.pallas import tpu as pltpu
from jax.experimental.pallas import tpu_sc as plsc

_N = 10000
_E = 320000
_D = 128
_ED = 16
_H = 256
_NC = 2
_NS = 16
_CH = 64
_IG = 8
_EP = 327680
_EPT = _EP // _NS
_NCHUNK = _EPT // _CH
_NP = 10240
_RPT = _NP // _NS
_WB = 16
_CB = 64


def _tables_body(x_ref, w_ref, o_ref):
    o_ref[...] = jnp.dot(x_ref[...], w_ref[...], preferred_element_type=jnp.float32)


def _node_tables(nf, Wab):
    return pl.pallas_call(
        _tables_body,
        grid=(4,),
        in_specs=[
            pl.BlockSpec((_N, _D), lambda j: (0, 0)),
            pl.BlockSpec((_D, _D), lambda j: (0, j)),
        ],
        out_specs=pl.BlockSpec((_N, _D), lambda j: (j, 0)),
        out_shape=jax.ShapeDtypeStruct((4 * _N, _D), jnp.float32),
    )(nf, Wab)


_BE = 4096


def _q_body(ef_ref, w_ref, b_ref, o_ref):
    o_ref[0] = (
        jnp.dot(ef_ref[...], w_ref[0], preferred_element_type=jnp.float32)
        + b_ref[0]
    )


def _edge_q(ef, W1c3, b13):
    return pl.pallas_call(
        _q_body,
        grid=(2, _EP // _BE),
        in_specs=[
            pl.BlockSpec((_BE, _ED), lambda j, i: (i, 0)),
            pl.BlockSpec((1, _ED, _D), lambda j, i: (j, 0, 0)),
            pl.BlockSpec((1, 1, _D), lambda j, i: (j, 0, 0)),
        ],
        out_specs=pl.BlockSpec((1, _BE, _D), lambda j, i: (j, i, 0)),
        out_shape=jax.ShapeDtypeStruct((2, _EP, _D), jnp.float32),
    )(ef, W1c3, b13)


def _sum_body(pp4, q2, srcr, dstr, s_out,
              srcc, dstc, sgbuf, dgbuf, dsbuf, bufA, bufB, bufQ, wbuf,
              s_sh, semA, semB, semQ):
    cid = lax.axis_index("c")
    sid = lax.axis_index("s")
    offA = cid * _N
    offB = 2 * _N + cid * _N

    zero = jnp.zeros((16,), jnp.float32)

    def zf(i, carry):
        for c in range(_D // 16):
            wbuf[i, pl.ds(c * 16, 16)] = zero
        return carry

    lax.fori_loop(0, _WB, zf, 0)
    for q in range(_RPT // _WB):
        pltpu.sync_copy(wbuf, s_sh.at[pl.ds(sid * _RPT + q * _WB, _WB)])
    plsc.subcore_barrier()

    qrowbase = cid * _EP + sid * _EPT
    idxblockbase = sid * (_NCHUNK // _IG)
    nclamp = _N - 1

    def outer(m, carry):
        pltpu.sync_copy(srcr.at[idxblockbase + m], srcc)
        pltpu.sync_copy(dstr.at[idxblockbase + m], dstc)

        def body(g, c1):
            j = m * _IG + g
            for c in range(_CH // 16):
                s = pl.ds(c * 16, 16)
                sgbuf[s] = srcc[g, s] + offA
                dgbuf[s] = jnp.minimum(dstc[g, s], nclamp) + offB
                dsbuf[s] = dstc[g, s]
            ca = pltpu.async_copy(pp4.at[sgbuf], bufA, semA)
            cb = pltpu.async_copy(pp4.at[dgbuf], bufB, semB)
            cq = pltpu.async_copy(
                q2.at[pl.ds(qrowbase + j * _CH, _CH)], bufQ, semQ)
            ca.wait()
            cb.wait()
            cq.wait()

            def comp(i, c2):
                for c in range(_D // 16):
                    s = pl.ds(c * 16, 16)
                    bufA[i, s] = jnp.maximum(
                        bufA[i, s] + bufB[i, s] + bufQ[i, s], 0.0)
                return c2

            lax.fori_loop(0, _CH, comp, 0)
            pltpu.sync_copy(bufA, s_sh.at[dsbuf], add=True)
            return c1

        lax.fori_loop(0, _IG, body, 0)
        return carry

    lax.fori_loop(0, _NCHUNK // _IG, outer, 0)
    plsc.subcore_barrier()

    for q in range(_RPT // _WB):
        pltpu.sync_copy(s_sh.at[pl.ds(sid * _RPT + q * _WB, _WB)], wbuf)
        pltpu.sync_copy(
            wbuf, s_out.at[pl.ds(cid * _NP + sid * _RPT + q * _WB, _WB)])


def _cnt_body(dstr, cnt_out, dstc, dsbuf, ones_v, cnt_sh):
    cid = lax.axis_index("c")
    sid = lax.axis_index("s")
    wid = sid * _NC + cid

    zero = jnp.zeros((16,), jnp.float32)
    one = jnp.ones((16,), jnp.float32)

    def zc(i, carry):
        for c in range(_D // 16):
            ones_v[i, pl.ds(c * 16, 16)] = zero
        return carry

    lax.fori_loop(0, _CB, zc, 0)
    for q in range(_RPT // _CB):
        pltpu.sync_copy(ones_v, cnt_sh.at[pl.ds(sid * _RPT + q * _CB, _CB)])

    def of(i, carry):
        ones_v[i, pl.ds(0, 16)] = one
        return carry

    lax.fori_loop(0, _CB, of, 0)
    plsc.subcore_barrier()

    nblocks = _EP // (_IG * _CH) // (_NC * _NS)
    blockbase = wid * nblocks

    def outer(m, carry):
        pltpu.sync_copy(dstr.at[blockbase + m], dstc)

        def body(g, c1):
            for c in range(_CH // 16):
                s = pl.ds(c * 16, 16)
                dsbuf[s] = dstc[g, s]
            pltpu.sync_copy(ones_v, cnt_sh.at[dsbuf], add=True)
            return c1

        lax.fori_loop(0, _IG, body, 0)
        return carry

    lax.fori_loop(0, nblocks, outer, 0)
    plsc.subcore_barrier()

    for q in range(_RPT // _CB):
        pltpu.sync_copy(cnt_sh.at[pl.ds(sid * _RPT + q * _CB, _CB)], ones_v)
        pltpu.sync_copy(
            ones_v, cnt_out.at[pl.ds(cid * _NP + sid * _RPT + q * _CB, _CB)])


def _sc_mesh():
    return plsc.VectorSubcoreMesh(core_axis_name="c", subcore_axis_name="s")


def _sc_sum(pp4, q2, src2, dst2):
    return pl.kernel(
        _sum_body,
        out_type=jax.ShapeDtypeStruct((2 * _NP, _D), jnp.float32),
        mesh=_sc_mesh(),
        scratch_types=[
            pltpu.VMEM((_IG, _CH), jnp.int32),
            pltpu.VMEM((_IG, _CH), jnp.int32),
            pltpu.VMEM((_CH,), jnp.int32),
            pltpu.VMEM((_CH,), jnp.int32),
            pltpu.VMEM((_CH,), jnp.int32),
            pltpu.VMEM((_CH, _D), jnp.float32),
            pltpu.VMEM((_CH, _D), jnp.float32),
            pltpu.VMEM((_CH, _D), jnp.float32),
            pltpu.VMEM((_WB, _D), jnp.float32),
            pltpu.VMEM_SHARED((_NP, _D), jnp.float32),
            pltpu.SemaphoreType.DMA,
            pltpu.SemaphoreType.DMA,
            pltpu.SemaphoreType.DMA,
        ],
    )(pp4, q2, src2, dst2)


def _sc_cnt(dst2):
    return pl.kernel(
        _cnt_body,
        out_type=jax.ShapeDtypeStruct((2 * _NP, _D), jnp.float32),
        mesh=_sc_mesh(),
        scratch_types=[
            pltpu.VMEM((_IG, _CH), jnp.int32),
            pltpu.VMEM((_CH,), jnp.int32),
            pltpu.VMEM((_CB, _D), jnp.float32),
            pltpu.VMEM_SHARED((_NP, _D), jnp.float32),
        ],
    )(dst2)


_RB = 2000


def _gru_body(s0_ref, s1_ref, cnta_ref, cntb_ref, h_ref, w2_ref, wih_ref,
              whh_ref, b2_ref, bih_ref, bhh_ref, o_ref):
    cnt = cnta_ref[...][:, 0:1] + cntb_ref[...][:, 0:1]
    inv = 1.0 / jnp.maximum(cnt, 1.0)
    pres = cnt * inv
    w2 = w2_ref[...]
    aggs = (
        jnp.dot(s0_ref[...], w2[:_D], preferred_element_type=jnp.float32)
        + jnp.dot(s1_ref[...], w2[_D:], preferred_element_type=jnp.float32)
    )
    agg = aggs * inv + b2_ref[...] * pres
    dn = (((1,), (1,)), ((), ()))
    gi = lax.dot_general(agg, wih_ref[...], dn,
                         preferred_element_type=jnp.float32) + bih_ref[...]
    h = h_ref[...]
    gh = lax.dot_general(h, whh_ref[...], dn,
                         preferred_element_type=jnp.float32) + bhh_ref[...]
    r = jax.nn.sigmoid(gi[:, :_D] + gh[:, :_D])
    z = jax.nn.sigmoid(gi[:, _D:2 * _D] + gh[:, _D:2 * _D])
    nn = jnp.tanh(gi[:, 2 * _D:] + r * gh[:, 2 * _D:])
    o_ref[...] = (1.0 - z) * nn + z * h


def _gru_update(s0, s1, cnta, cntb, nf, W2, b2, Wih, Whh, bih, bhh):
    return pl.pallas_call(
        _gru_body,
        grid=(_N // _RB,),
        in_specs=[
            pl.BlockSpec((_RB, _D), lambda i: (i, 0)),
            pl.BlockSpec((_RB, _D), lambda i: (i, 0)),
            pl.BlockSpec((_RB, _D), lambda i: (i, 0)),
            pl.BlockSpec((_RB, _D), lambda i: (i, 0)),
            pl.BlockSpec((_RB, _D), lambda i: (i, 0)),
            pl.BlockSpec((_H, _H), lambda i: (0, 0)),
            pl.BlockSpec((3 * _D, _H), lambda i: (0, 0)),
            pl.BlockSpec((3 * _D, _D), lambda i: (0, 0)),
            pl.BlockSpec((1, _H), lambda i: (0, 0)),
            pl.BlockSpec((1, 3 * _D), lambda i: (0, 0)),
            pl.BlockSpec((1, 3 * _D), lambda i: (0, 0)),
        ],
        out_specs=pl.BlockSpec((_RB, _D), lambda i: (i, 0)),
        out_shape=jax.ShapeDtypeStruct((_N, _D), jnp.float32),
    )(s0, s1, cnta, cntb, nf, W2, Wih, Whh, b2.reshape(1, _H),
      bih.reshape(1, 3 * _D), bhh.reshape(1, 3 * _D))


def kernel(node_feats, edge_feats, W1, b1, W2, b2, Wih, Whh, bih, bhh,
           edge_index):
    nf = node_feats
    Wab = jnp.concatenate([W1[:_D], W1[_D:2 * _D]], axis=1)
    W1c3 = W1[2 * _D:].reshape(_ED, 2, _D).transpose(1, 0, 2)
    b13 = b1.reshape(2, 1, _D)
    pad = _EP - _E
    src2 = jnp.concatenate(
        [edge_index[0].astype(jnp.int32), jnp.zeros((pad,), jnp.int32)]
    ).reshape(_EP // (_IG * _CH), _IG, _CH)
    dst2 = jnp.concatenate(
        [edge_index[1].astype(jnp.int32), jnp.full((pad,), _N, jnp.int32)]
    ).reshape(_EP // (_IG * _CH), _IG, _CH)
    ef_p = jnp.concatenate(
        [edge_feats, jnp.zeros((pad, _ED), jnp.float32)], axis=0)

    pp4 = _node_tables(nf, Wab)
    q2 = _edge_q(ef_p, W1c3, b13).reshape(2 * _EP, _D)
    s2 = _sc_sum(pp4, q2, src2, dst2)
    cnt16 = _sc_cnt(dst2)
    return _gru_update(s2[:_N], s2[_NP:_NP + _N],
                       cnt16[:_N], cnt16[_NP:_NP + _N], nf,
                       W2, b2, Wih, Whh, bih, bhh)
